# baseline (speedup 1.0000x reference)
import jax
import jax.numpy as jnp
from jax import lax
from jax.experimental import pallas as pl
from jax.experimental.pallas import tpu as pltpu
from jax.experimental.pallas import tpu_sc as plsc

NUM_FACTORS = 64
NUM_ROWS = 100000
BATCH = 16384
L = 16
NC = 2
NS = 16
NW = NC * NS
B_PER_W = BATCH // NW
CHUNK = 256
N_CHUNKS = B_PER_W // CHUNK
BLK = 128
N_FULL_BLK = NUM_ROWS // BLK
LAST_FULL = N_FULL_BLK - 1
BLK_PER_W = 25
ROW = 2 * NUM_FACTORS
TAIL = NUM_ROWS - N_FULL_BLK * BLK

_COMPILER_PARAMS = pltpu.CompilerParams(
    needs_layout_passes=False, use_tc_tiling_on_sc=True)


def _transpose_kernel(uft_hbm, ift_hbm, ulin_hbm, ilin_hbm,
                      ub0, ub1, ib0, ib1, su0, su1, si0, si1,
                      dsu0, dsu1, dsi0, dsi1, wsu0, wsu1, wsi0, wsi1):
    wid = lax.axis_index("s") * NC + lax.axis_index("c")
    ridx = [lax.iota(jnp.int32, L) + q * L for q in range(4)]

    def transpose_block(src_u, src_i, dst_u, dst_i, ncc):
        def tb(cc, carry):
            cbase = jnp.broadcast_to(cc * 4, (L,)).astype(jnp.int32)
            for u in range(4):
                csplat = cbase + u
                row = 2 * cc + (u >> 1)
                half = (u & 1) * NUM_FACTORS
                for q in range(4):
                    vu = plsc.load_gather(src_u, [ridx[q], csplat])
                    dst_u[row, pl.ds(half + q * L, L)] = vu
                    vi = plsc.load_gather(src_i, [ridx[q], csplat])
                    dst_i[row, pl.ds(half + q * L, L)] = vi
            return carry
        lax.fori_loop(0, ncc, tb, 0)

    def do_block(j, ub_c, ib_c, su_c, si_c, dsem_u, dsem_i, wsem_u, wsem_i,
                 ub_n, ib_n, dsem_un, dsem_in):
        k = jnp.minimum(wid + 32 * j, LAST_FULL)

        @pl.when(j < BLK_PER_W - 1)
        def _prefetch():
            kn = jnp.minimum(wid + 32 * (j + 1), LAST_FULL)
            pltpu.async_copy(uft_hbm.at[:, pl.ds(kn * BLK, BLK)], ub_n,
                             dsem_un)
            pltpu.async_copy(ift_hbm.at[:, pl.ds(kn * BLK, BLK)], ib_n,
                             dsem_in)

        pltpu.make_async_copy(uft_hbm.at[:, pl.ds(0, BLK)], ub_c,
                              dsem_u).wait()
        pltpu.make_async_copy(ift_hbm.at[:, pl.ds(0, BLK)], ib_c,
                              dsem_i).wait()

        @pl.when(j >= 2)
        def _drain():
            pltpu.make_async_copy(su_c, ulin_hbm.at[pl.ds(0, 64), :],
                                  wsem_u).wait()
            pltpu.make_async_copy(si_c, ilin_hbm.at[pl.ds(0, 64), :],
                                  wsem_i).wait()

        transpose_block(ub_c, ib_c, su_c, si_c, BLK // 4)

        r0 = k * (BLK // 2)
        pltpu.async_copy(su_c, ulin_hbm.at[pl.ds(r0, 64), :], wsem_u)
        pltpu.async_copy(si_c, ilin_hbm.at[pl.ds(r0, 64), :], wsem_i)

    k0 = wid
    pltpu.async_copy(uft_hbm.at[:, pl.ds(k0 * BLK, BLK)], ub0, dsu0)
    pltpu.async_copy(ift_hbm.at[:, pl.ds(k0 * BLK, BLK)], ib0, dsi0)

    def body(j, carry):
        @pl.when((j & 1) == 0)
        def _even():
            do_block(j, ub0, ib0, su0, si0, dsu0, dsi0, wsu0, wsi0,
                     ub1, ib1, dsu1, dsi1)

        @pl.when((j & 1) == 1)
        def _odd():
            do_block(j, ub1, ib1, su1, si1, dsu1, dsi1, wsu1, wsi1,
                     ub0, ib0, dsu0, dsi0)
        return carry

    lax.fori_loop(0, BLK_PER_W, body, 0)

    pltpu.make_async_copy(su0, ulin_hbm.at[pl.ds(0, 64), :], wsu0).wait()
    pltpu.make_async_copy(si0, ilin_hbm.at[pl.ds(0, 64), :], wsi0).wait()
    pltpu.make_async_copy(su1, ulin_hbm.at[pl.ds(0, 64), :], wsu1).wait()
    pltpu.make_async_copy(si1, ilin_hbm.at[pl.ds(0, 64), :], wsi1).wait()


def _gather_kernel(uid_hbm, iid_hbm, uf_hbm, if_hbm, w_hbm, bias_hbm, out_hbm,
                   uidx_v, iidx_v, uridx_v, iridx_v, urows_v, irows_v, w_v,
                   bias_v, out_v, sem_u, sem_i):
    wid = lax.axis_index("s") * NC + lax.axis_index("c")
    base = wid * B_PER_W

    pltpu.sync_copy(uid_hbm.at[pl.ds(base, B_PER_W)], uidx_v)
    pltpu.sync_copy(iid_hbm.at[pl.ds(base, B_PER_W)], iidx_v)
    pltpu.sync_copy(w_hbm, w_v)
    pltpu.sync_copy(bias_hbm, bias_v)

    w0 = w_v[pl.ds(0, L)]
    w1 = w_v[pl.ds(L, L)]
    w2 = w_v[pl.ds(2 * L, L)]
    w3 = w_v[pl.ds(3 * L, L)]
    bias = bias_v[...]
    iota = lax.iota(jnp.int32, L)
    lane_masks = [iota == j for j in range(L)]

    def chunk_body(c, carry):
        c0 = c * CHUNK
        for v in range(CHUNK // L):
            uvec = uidx_v[pl.ds(c0 + v * L, L)]
            uridx_v[pl.ds(v * L, L)] = lax.shift_right_logical(uvec, 1)
        cu = pltpu.async_copy(uf_hbm.at[uridx_v], urows_v, sem_u)
        for v in range(CHUNK // L):
            ivec = iidx_v[pl.ds(c0 + v * L, L)]
            iridx_v[pl.ds(v * L, L)] = lax.shift_right_logical(ivec, 1)
        ci = pltpu.async_copy(if_hbm.at[iridx_v], irows_v, sem_i)
        cu.wait()
        ci.wait()

        def group_body(g, carry2):
            b0 = g * L
            uvec = uidx_v[pl.ds(c0 + b0, L)]
            ivec = iidx_v[pl.ds(c0 + b0, L)]
            upar = (uvec & 1) * NUM_FACTORS
            ipar = (ivec & 1) * NUM_FACTORS
            acc = bias
            for j in range(L):
                b = b0 + j
                pu = upar[j]
                pi = ipar[j]
                t = (urows_v[b, pl.ds(pu, L)] * irows_v[b, pl.ds(pi, L)] * w0
                     + urows_v[b, pl.ds(pu + L, L)]
                     * irows_v[b, pl.ds(pi + L, L)] * w1
                     + urows_v[b, pl.ds(pu + 2 * L, L)]
                     * irows_v[b, pl.ds(pi + 2 * L, L)] * w2
                     + urows_v[b, pl.ds(pu + 3 * L, L)]
                     * irows_v[b, pl.ds(pi + 3 * L, L)] * w3)
                acc = jnp.where(lane_masks[j], acc + jnp.sum(t), acc)
            out_v[pl.ds(c0 + b0, L)] = acc
            return carry2

        lax.fori_loop(0, CHUNK // L, group_body, 0)
        return carry

    lax.fori_loop(0, N_CHUNKS, chunk_body, 0)

    pltpu.sync_copy(out_v, out_hbm.at[pl.ds(base, B_PER_W)])


@jax.jit
def _run(user_ids, item_ids, user_factors, item_factors, w_vec, bias_splat):
    mesh = plsc.VectorSubcoreMesh(core_axis_name="c", subcore_axis_name="s")
    uft = user_factors.T
    ift = item_factors.T

    tfn = pl.kernel(
        _transpose_kernel,
        mesh=mesh,
        compiler_params=_COMPILER_PARAMS,
        out_type=(
            jax.ShapeDtypeStruct((NUM_ROWS // 2, ROW), jnp.float32),
            jax.ShapeDtypeStruct((NUM_ROWS // 2, ROW), jnp.float32),
        ),
        scratch_types=(
            [pltpu.VMEM((NUM_FACTORS, BLK), jnp.float32)] * 4
            + [pltpu.VMEM((BLK // 2, ROW), jnp.float32)] * 4
            + [pltpu.SemaphoreType.DMA] * 8
        ),
    )
    ulin, ilin = tfn(uft, ift)

    s = N_FULL_BLK * BLK
    utail = lax.slice(user_factors, (s, 0), (NUM_ROWS, NUM_FACTORS))
    itail = lax.slice(item_factors, (s, 0), (NUM_ROWS, NUM_FACTORS))
    ulin = lax.dynamic_update_slice(ulin, utail.reshape(TAIL // 2, ROW),
                                    (s // 2, 0))
    ilin = lax.dynamic_update_slice(ilin, itail.reshape(TAIL // 2, ROW),
                                    (s // 2, 0))

    gfn = pl.kernel(
        _gather_kernel,
        mesh=mesh,
        compiler_params=_COMPILER_PARAMS,
        out_type=jax.ShapeDtypeStruct((BATCH,), jnp.float32),
        scratch_types=[
            pltpu.VMEM((B_PER_W,), jnp.int32),
            pltpu.VMEM((B_PER_W,), jnp.int32),
            pltpu.VMEM((CHUNK,), jnp.int32),
            pltpu.VMEM((CHUNK,), jnp.int32),
            pltpu.VMEM((CHUNK, ROW), jnp.float32),
            pltpu.VMEM((CHUNK, ROW), jnp.float32),
            pltpu.VMEM((NUM_FACTORS,), jnp.float32),
            pltpu.VMEM((L,), jnp.float32),
            pltpu.VMEM((B_PER_W,), jnp.float32),
            pltpu.SemaphoreType.DMA,
            pltpu.SemaphoreType.DMA,
        ],
    )
    return gfn(user_ids, item_ids, ulin, ilin, w_vec, bias_splat)


def kernel(user_ids, item_ids, user_factors, item_factors, W, b):
    uid = user_ids.astype(jnp.int32)
    iid = item_ids.astype(jnp.int32)
    w_vec = W.reshape(NUM_FACTORS).astype(jnp.float32)
    bias_splat = jnp.broadcast_to(b.astype(jnp.float32), (L,))
    out = _run(uid, iid, user_factors, item_factors, w_vec, bias_splat)
    return out.reshape(BATCH, 1)


# speedup vs baseline: 1.2958x; 1.2958x over previous
import jax
import jax.numpy as jnp
from jax import lax
from jax.experimental import pallas as pl
from jax.experimental.pallas import tpu as pltpu
from jax.experimental.pallas import tpu_sc as plsc

NUM_FACTORS = 64
NUM_ROWS = 100000
BATCH = 16384
L = 16
NC = 2
NS = 16
NW = NC * NS
B_PER_W = BATCH // NW
CHUNK = 256
N_CHUNKS = B_PER_W // CHUNK
BLK = 128
N_FULL_BLK = NUM_ROWS // BLK
LAST_FULL = N_FULL_BLK - 1
BLK_PER_W = 25
ROW = 2 * NUM_FACTORS
TAIL = NUM_ROWS - N_FULL_BLK * BLK

_COMPILER_PARAMS = pltpu.CompilerParams(
    needs_layout_passes=False, use_tc_tiling_on_sc=True)


def _transpose_kernel(uft_hbm, ift_hbm, ulin_hbm, ilin_hbm,
                      ub0, ub1, ib0, ib1, su0, su1, si0, si1,
                      dsu0, dsu1, dsi0, dsi1, wsu0, wsu1, wsi0, wsi1):
    wid = lax.axis_index("s") * NC + lax.axis_index("c")
    ridx = [lax.iota(jnp.int32, L) + q * L for q in range(4)]

    def transpose_block(src_u, src_i, dst_u, dst_i, ncols):
        @plsc.parallel_loop(0, ncols, unroll=2)
        def _col(c):
            csplat = jnp.broadcast_to(c, (L,)).astype(jnp.int32)
            row = lax.shift_right_logical(c, 1)
            half = (c & 1) * NUM_FACTORS
            vu = [plsc.load_gather(src_u, [ridx[q], csplat])
                  for q in range(4)]
            vi = [plsc.load_gather(src_i, [ridx[q], csplat])
                  for q in range(4)]
            for q in range(4):
                dst_u[row, pl.ds(half + q * L, L)] = vu[q]
            for q in range(4):
                dst_i[row, pl.ds(half + q * L, L)] = vi[q]

    def do_block(j, ub_c, ib_c, su_c, si_c, dsem_u, dsem_i, wsem_u, wsem_i,
                 ub_n, ib_n, dsem_un, dsem_in):
        k = jnp.minimum(wid + 32 * j, LAST_FULL)

        @pl.when(j < BLK_PER_W - 1)
        def _prefetch():
            kn = jnp.minimum(wid + 32 * (j + 1), LAST_FULL)
            pltpu.async_copy(uft_hbm.at[:, pl.ds(kn * BLK, BLK)], ub_n,
                             dsem_un)
            pltpu.async_copy(ift_hbm.at[:, pl.ds(kn * BLK, BLK)], ib_n,
                             dsem_in)

        pltpu.make_async_copy(uft_hbm.at[:, pl.ds(0, BLK)], ub_c,
                              dsem_u).wait()
        pltpu.make_async_copy(ift_hbm.at[:, pl.ds(0, BLK)], ib_c,
                              dsem_i).wait()

        @pl.when(j >= 2)
        def _drain():
            pltpu.make_async_copy(su_c, ulin_hbm.at[pl.ds(0, 64), :],
                                  wsem_u).wait()
            pltpu.make_async_copy(si_c, ilin_hbm.at[pl.ds(0, 64), :],
                                  wsem_i).wait()

        transpose_block(ub_c, ib_c, su_c, si_c, BLK)

        r0 = k * (BLK // 2)
        pltpu.async_copy(su_c, ulin_hbm.at[pl.ds(r0, 64), :], wsem_u)
        pltpu.async_copy(si_c, ilin_hbm.at[pl.ds(r0, 64), :], wsem_i)

    k0 = wid
    pltpu.async_copy(uft_hbm.at[:, pl.ds(k0 * BLK, BLK)], ub0, dsu0)
    pltpu.async_copy(ift_hbm.at[:, pl.ds(k0 * BLK, BLK)], ib0, dsi0)

    def body(j, carry):
        @pl.when((j & 1) == 0)
        def _even():
            do_block(j, ub0, ib0, su0, si0, dsu0, dsi0, wsu0, wsi0,
                     ub1, ib1, dsu1, dsi1)

        @pl.when((j & 1) == 1)
        def _odd():
            do_block(j, ub1, ib1, su1, si1, dsu1, dsi1, wsu1, wsi1,
                     ub0, ib0, dsu0, dsi0)
        return carry

    lax.fori_loop(0, BLK_PER_W, body, 0)

    pltpu.make_async_copy(su0, ulin_hbm.at[pl.ds(0, 64), :], wsu0).wait()
    pltpu.make_async_copy(si0, ilin_hbm.at[pl.ds(0, 64), :], wsi0).wait()
    pltpu.make_async_copy(su1, ulin_hbm.at[pl.ds(0, 64), :], wsu1).wait()
    pltpu.make_async_copy(si1, ilin_hbm.at[pl.ds(0, 64), :], wsi1).wait()


def _gather_kernel(uid_hbm, iid_hbm, uf_hbm, if_hbm, w_hbm, bias_hbm, out_hbm,
                   uidx_v, iidx_v, uridx_v, iridx_v, urows_v, irows_v, w_v,
                   bias_v, out_v, sem_u, sem_i):
    wid = lax.axis_index("s") * NC + lax.axis_index("c")
    base = wid * B_PER_W

    pltpu.sync_copy(uid_hbm.at[pl.ds(base, B_PER_W)], uidx_v)
    pltpu.sync_copy(iid_hbm.at[pl.ds(base, B_PER_W)], iidx_v)
    pltpu.sync_copy(w_hbm, w_v)
    pltpu.sync_copy(bias_hbm, bias_v)

    w0 = w_v[pl.ds(0, L)]
    w1 = w_v[pl.ds(L, L)]
    w2 = w_v[pl.ds(2 * L, L)]
    w3 = w_v[pl.ds(3 * L, L)]
    bias = bias_v[...]
    iota = lax.iota(jnp.int32, L)
    lane_masks = [iota == j for j in range(L)]

    def chunk_body(c, carry):
        c0 = c * CHUNK
        for v in range(CHUNK // L):
            uvec = uidx_v[pl.ds(c0 + v * L, L)]
            uridx_v[pl.ds(v * L, L)] = lax.shift_right_logical(uvec, 1)
        cu = pltpu.async_copy(uf_hbm.at[uridx_v], urows_v, sem_u)
        for v in range(CHUNK // L):
            ivec = iidx_v[pl.ds(c0 + v * L, L)]
            iridx_v[pl.ds(v * L, L)] = lax.shift_right_logical(ivec, 1)
        ci = pltpu.async_copy(if_hbm.at[iridx_v], irows_v, sem_i)
        cu.wait()
        ci.wait()

        def group_body(g, carry2):
            b0 = g * L
            uvec = uidx_v[pl.ds(c0 + b0, L)]
            ivec = iidx_v[pl.ds(c0 + b0, L)]
            upar = (uvec & 1) * NUM_FACTORS
            ipar = (ivec & 1) * NUM_FACTORS
            acc = bias
            for j in range(L):
                b = b0 + j
                pu = upar[j]
                pi = ipar[j]
                t = (urows_v[b, pl.ds(pu, L)] * irows_v[b, pl.ds(pi, L)] * w0
                     + urows_v[b, pl.ds(pu + L, L)]
                     * irows_v[b, pl.ds(pi + L, L)] * w1
                     + urows_v[b, pl.ds(pu + 2 * L, L)]
                     * irows_v[b, pl.ds(pi + 2 * L, L)] * w2
                     + urows_v[b, pl.ds(pu + 3 * L, L)]
                     * irows_v[b, pl.ds(pi + 3 * L, L)] * w3)
                acc = jnp.where(lane_masks[j], acc + jnp.sum(t), acc)
            out_v[pl.ds(c0 + b0, L)] = acc
            return carry2

        lax.fori_loop(0, CHUNK // L, group_body, 0)
        return carry

    lax.fori_loop(0, N_CHUNKS, chunk_body, 0)

    pltpu.sync_copy(out_v, out_hbm.at[pl.ds(base, B_PER_W)])


@jax.jit
def _run(user_ids, item_ids, user_factors, item_factors, w_vec, bias_splat):
    mesh = plsc.VectorSubcoreMesh(core_axis_name="c", subcore_axis_name="s")
    uft = user_factors.T
    ift = item_factors.T

    tfn = pl.kernel(
        _transpose_kernel,
        mesh=mesh,
        compiler_params=_COMPILER_PARAMS,
        out_type=(
            jax.ShapeDtypeStruct((NUM_ROWS // 2, ROW), jnp.float32),
            jax.ShapeDtypeStruct((NUM_ROWS // 2, ROW), jnp.float32),
        ),
        scratch_types=(
            [pltpu.VMEM((NUM_FACTORS, BLK), jnp.float32)] * 4
            + [pltpu.VMEM((BLK // 2, ROW), jnp.float32)] * 4
            + [pltpu.SemaphoreType.DMA] * 8
        ),
    )
    ulin, ilin = tfn(uft, ift)

    s = N_FULL_BLK * BLK
    utail = lax.slice(user_factors, (s, 0), (NUM_ROWS, NUM_FACTORS))
    itail = lax.slice(item_factors, (s, 0), (NUM_ROWS, NUM_FACTORS))
    ulin = lax.dynamic_update_slice(ulin, utail.reshape(TAIL // 2, ROW),
                                    (s // 2, 0))
    ilin = lax.dynamic_update_slice(ilin, itail.reshape(TAIL // 2, ROW),
                                    (s // 2, 0))

    gfn = pl.kernel(
        _gather_kernel,
        mesh=mesh,
        compiler_params=_COMPILER_PARAMS,
        out_type=jax.ShapeDtypeStruct((BATCH,), jnp.float32),
        scratch_types=[
            pltpu.VMEM((B_PER_W,), jnp.int32),
            pltpu.VMEM((B_PER_W,), jnp.int32),
            pltpu.VMEM((CHUNK,), jnp.int32),
            pltpu.VMEM((CHUNK,), jnp.int32),
            pltpu.VMEM((CHUNK, ROW), jnp.float32),
            pltpu.VMEM((CHUNK, ROW), jnp.float32),
            pltpu.VMEM((NUM_FACTORS,), jnp.float32),
            pltpu.VMEM((L,), jnp.float32),
            pltpu.VMEM((B_PER_W,), jnp.float32),
            pltpu.SemaphoreType.DMA,
            pltpu.SemaphoreType.DMA,
        ],
    )
    return gfn(user_ids, item_ids, ulin, ilin, w_vec, bias_splat)


def kernel(user_ids, item_ids, user_factors, item_factors, W, b):
    uid = user_ids.astype(jnp.int32)
    iid = item_ids.astype(jnp.int32)
    w_vec = W.reshape(NUM_FACTORS).astype(jnp.float32)
    bias_splat = jnp.broadcast_to(b.astype(jnp.float32), (L,))
    out = _run(uid, iid, user_factors, item_factors, w_vec, bias_splat)
    return out.reshape(BATCH, 1)


# speedup vs baseline: 4.7640x; 3.6764x over previous
import jax
import jax.numpy as jnp
from jax import lax
from jax.experimental import pallas as pl
from jax.experimental.pallas import tpu as pltpu
from jax.experimental.pallas import tpu_sc as plsc

NUM_FACTORS = 64
NUM_ROWS = 100000
BATCH = 16384
L = 16
NC = 2
NS = 16
NW = NC * NS
B_PER_W = BATCH // NW
MAIN = 99968
TAILPAD = 128
ROWBUF = MAIN + TAILPAD
HALF = BATCH // 2

_COMPILER_PARAMS = pltpu.CompilerParams(
    needs_layout_passes=False, use_tc_tiling_on_sc=True)


def _pass1_kernel(uft_hbm, ift_hbm, utail_hbm, itail_hbm, uid_hbm, iid_hbm,
                  w_hbm, upart_hbm, ipart_hbm,
                  rowbuf, ids_v, stage_v, w_v, sem_m, sem_t):
    wid = lax.axis_index("s") * NC + lax.axis_index("c")
    pltpu.sync_copy(w_hbm, w_v)

    def unit(f, table, tail, ids, part, scale):
        cm = pltpu.async_copy(table.at[f, pl.ds(0, MAIN)],
                              rowbuf.at[pl.ds(0, MAIN)], sem_m)
        ct = pltpu.async_copy(tail.at[f, :],
                              rowbuf.at[pl.ds(MAIN, TAILPAD)], sem_t)
        pltpu.sync_copy(ids.at[pl.ds(0, HALF)], ids_v)
        cm.wait()
        ct.wait()
        wspl = plsc.load_gather(w_v, [jnp.broadcast_to(f, (L,))])
        for h in range(2):
            if h == 1:
                pltpu.sync_copy(ids.at[pl.ds(HALF, HALF)], ids_v)

            @plsc.parallel_loop(0, HALF // L, unroll=4)
            def _g(v):
                idxv = ids_v[pl.ds(v * L, L)]
                val = plsc.load_gather(rowbuf, [idxv])
                if scale:
                    val = val * wspl
                stage_v[pl.ds(v * L, L)] = val

            pltpu.sync_copy(stage_v, part.at[f, pl.ds(h * HALF, HALF)])

    unit(wid, uft_hbm, utail_hbm, uid_hbm, upart_hbm, True)
    unit(wid + 32, uft_hbm, utail_hbm, uid_hbm, upart_hbm, True)
    unit(wid, ift_hbm, itail_hbm, iid_hbm, ipart_hbm, False)
    unit(wid + 32, ift_hbm, itail_hbm, iid_hbm, ipart_hbm, False)


def _pass2_kernel(upart_hbm, ipart_hbm, bias_hbm, out_hbm,
                  us_v, is_v, bias_v, out_v, sem_u, sem_i):
    wid = lax.axis_index("s") * NC + lax.axis_index("c")
    base = wid * B_PER_W
    cu = pltpu.async_copy(upart_hbm.at[:, pl.ds(base, B_PER_W)], us_v, sem_u)
    ci = pltpu.async_copy(ipart_hbm.at[:, pl.ds(base, B_PER_W)], is_v, sem_i)
    pltpu.sync_copy(bias_hbm, bias_v)
    cu.wait()
    ci.wait()
    bias = bias_v[...]

    def slice_body(sl, carry):
        s0 = sl * L
        acc = bias
        for f in range(NUM_FACTORS):
            acc = acc + us_v[f, pl.ds(s0, L)] * is_v[f, pl.ds(s0, L)]
        out_v[pl.ds(s0, L)] = acc
        return carry

    lax.fori_loop(0, B_PER_W // L, slice_body, 0)
    pltpu.sync_copy(out_v, out_hbm.at[pl.ds(base, B_PER_W)])


@jax.jit
def _run(user_ids, item_ids, user_factors, item_factors, w_vec, bias_splat):
    mesh = plsc.VectorSubcoreMesh(core_axis_name="c", subcore_axis_name="s")
    uft = user_factors.T
    ift = item_factors.T
    utail = jnp.pad(lax.slice(uft, (0, MAIN), (NUM_FACTORS, NUM_ROWS)),
                    ((0, 0), (0, TAILPAD - (NUM_ROWS - MAIN))))
    itail = jnp.pad(lax.slice(ift, (0, MAIN), (NUM_FACTORS, NUM_ROWS)),
                    ((0, 0), (0, TAILPAD - (NUM_ROWS - MAIN))))

    p1 = pl.kernel(
        _pass1_kernel,
        mesh=mesh,
        compiler_params=_COMPILER_PARAMS,
        out_type=(
            jax.ShapeDtypeStruct((NUM_FACTORS, BATCH), jnp.float32),
            jax.ShapeDtypeStruct((NUM_FACTORS, BATCH), jnp.float32),
        ),
        scratch_types=[
            pltpu.VMEM((ROWBUF,), jnp.float32),
            pltpu.VMEM((HALF,), jnp.int32),
            pltpu.VMEM((HALF,), jnp.float32),
            pltpu.VMEM((NUM_FACTORS,), jnp.float32),
            pltpu.SemaphoreType.DMA,
            pltpu.SemaphoreType.DMA,
        ],
    )
    upart, ipart = p1(uft, ift, utail, itail, user_ids, item_ids, w_vec)

    p2 = pl.kernel(
        _pass2_kernel,
        mesh=mesh,
        compiler_params=_COMPILER_PARAMS,
        out_type=jax.ShapeDtypeStruct((BATCH,), jnp.float32),
        scratch_types=[
            pltpu.VMEM((NUM_FACTORS, B_PER_W), jnp.float32),
            pltpu.VMEM((NUM_FACTORS, B_PER_W), jnp.float32),
            pltpu.VMEM((L,), jnp.float32),
            pltpu.VMEM((B_PER_W,), jnp.float32),
            pltpu.SemaphoreType.DMA,
            pltpu.SemaphoreType.DMA,
        ],
    )
    return p2(upart, ipart, bias_splat)


def kernel(user_ids, item_ids, user_factors, item_factors, W, b):
    uid = user_ids.astype(jnp.int32)
    iid = item_ids.astype(jnp.int32)
    w_vec = W.reshape(NUM_FACTORS).astype(jnp.float32)
    bias_splat = jnp.broadcast_to(b.astype(jnp.float32), (L,))
    out = _run(uid, iid, user_factors, item_factors, w_vec, bias_splat)
    return out.reshape(BATCH, 1)


# speedup vs baseline: 4.8752x; 1.0234x over previous
import jax
import jax.numpy as jnp
from jax import lax
from jax.experimental import pallas as pl
from jax.experimental.pallas import tpu as pltpu
from jax.experimental.pallas import tpu_sc as plsc

NUM_FACTORS = 64
NUM_ROWS = 100000
BATCH = 16384
L = 16
NC = 2
NS = 16
NW = NC * NS
B_PER_W = BATCH // NW
MAIN = 99968
TAILPAD = 128
ROWBUF = MAIN + TAILPAD
CH = 4096
NCH = BATCH // CH

_COMPILER_PARAMS = pltpu.CompilerParams(
    needs_layout_passes=False, use_tc_tiling_on_sc=True)


def _pass1_kernel(uft_hbm, ift_hbm, utail_hbm, itail_hbm, uid_hbm, iid_hbm,
                  w_hbm, upart_hbm, ipart_hbm,
                  rowbuf, ids0_v, ids1_v, stage0_v, stage1_v, w_v,
                  sem_m, sem_t, sem_i0, sem_i1, sem_s0, sem_s1):
    wid = lax.axis_index("s") * NC + lax.axis_index("c")
    pltpu.sync_copy(w_hbm, w_v)

    idbufs = [(ids0_v, sem_i0), (ids1_v, sem_i1)]
    stages = [(stage0_v, sem_s0), (stage1_v, sem_s1)]
    gchunk = [0]

    def unit(f, table, tail, ids_hbm, part, scale):
        cm = pltpu.async_copy(table.at[f, pl.ds(0, MAIN)],
                              rowbuf.at[pl.ds(0, MAIN)], sem_m)
        ct = pltpu.async_copy(tail.at[f, :],
                              rowbuf.at[pl.ds(MAIN, TAILPAD)], sem_t)
        pltpu.async_copy(ids_hbm.at[pl.ds(0, CH)], idbufs[0][0],
                         idbufs[0][1])
        cm.wait()
        ct.wait()
        wspl = plsc.load_gather(w_v, [jnp.broadcast_to(f, (L,))])
        for h in range(NCH):
            ids_v, sem_i = idbufs[h % 2]
            stage_v, sem_s = stages[h % 2]
            if h + 1 < NCH:
                nxt_v, nxt_s = idbufs[(h + 1) % 2]
                pltpu.async_copy(ids_hbm.at[pl.ds((h + 1) * CH, CH)],
                                 nxt_v, nxt_s)
            pltpu.make_async_copy(ids_hbm.at[pl.ds(0, CH)], ids_v,
                                  sem_i).wait()
            if gchunk[0] >= 2:
                pltpu.make_async_copy(
                    stage_v, part.at[f, pl.ds(0, CH)], sem_s).wait()
            gchunk[0] += 1

            @plsc.parallel_loop(0, CH // L, unroll=4)
            def _g(v):
                idxv = ids_v[pl.ds(v * L, L)]
                val = plsc.load_gather(rowbuf, [idxv])
                if scale:
                    val = val * wspl
                stage_v[pl.ds(v * L, L)] = val

            pltpu.async_copy(stage_v, part.at[f, pl.ds(h * CH, CH)], sem_s)

    unit(wid, uft_hbm, utail_hbm, uid_hbm, upart_hbm, True)
    unit(wid + 32, uft_hbm, utail_hbm, uid_hbm, upart_hbm, True)
    unit(wid, ift_hbm, itail_hbm, iid_hbm, ipart_hbm, False)
    unit(wid + 32, ift_hbm, itail_hbm, iid_hbm, ipart_hbm, False)
    for stage_v, sem_s in stages:
        pltpu.make_async_copy(stage_v, ipart_hbm.at[0, pl.ds(0, CH)],
                              sem_s).wait()


def _pass2_kernel(upart_hbm, ipart_hbm, bias_hbm, out_hbm,
                  us_v, is_v, bias_v, out_v, sem_u0, sem_i0, sem_u1, sem_i1):
    wid = lax.axis_index("s") * NC + lax.axis_index("c")
    base = wid * B_PER_W
    hw = B_PER_W // 2
    cu0 = pltpu.async_copy(upart_hbm.at[:, pl.ds(base, hw)],
                           us_v.at[:, pl.ds(0, hw)], sem_u0)
    ci0 = pltpu.async_copy(ipart_hbm.at[:, pl.ds(base, hw)],
                           is_v.at[:, pl.ds(0, hw)], sem_i0)
    cu1 = pltpu.async_copy(upart_hbm.at[:, pl.ds(base + hw, hw)],
                           us_v.at[:, pl.ds(hw, hw)], sem_u1)
    ci1 = pltpu.async_copy(ipart_hbm.at[:, pl.ds(base + hw, hw)],
                           is_v.at[:, pl.ds(hw, hw)], sem_i1)
    pltpu.sync_copy(bias_hbm, bias_v)
    bias = bias_v[...]

    def slice_body(sl, carry):
        s0 = sl * L
        acc = bias
        for f in range(NUM_FACTORS):
            acc = acc + us_v[f, pl.ds(s0, L)] * is_v[f, pl.ds(s0, L)]
        out_v[pl.ds(s0, L)] = acc
        return carry

    cu0.wait()
    ci0.wait()
    lax.fori_loop(0, hw // L, slice_body, 0)
    cu1.wait()
    ci1.wait()
    lax.fori_loop(hw // L, B_PER_W // L, slice_body, 0)
    pltpu.sync_copy(out_v, out_hbm.at[pl.ds(base, B_PER_W)])


@jax.jit
def _run(user_ids, item_ids, user_factors, item_factors, w_vec, bias_splat):
    mesh = plsc.VectorSubcoreMesh(core_axis_name="c", subcore_axis_name="s")
    uft = user_factors.T
    ift = item_factors.T
    utail = jnp.pad(lax.slice(uft, (0, MAIN), (NUM_FACTORS, NUM_ROWS)),
                    ((0, 0), (0, TAILPAD - (NUM_ROWS - MAIN))))
    itail = jnp.pad(lax.slice(ift, (0, MAIN), (NUM_FACTORS, NUM_ROWS)),
                    ((0, 0), (0, TAILPAD - (NUM_ROWS - MAIN))))

    p1 = pl.kernel(
        _pass1_kernel,
        mesh=mesh,
        compiler_params=_COMPILER_PARAMS,
        out_type=(
            jax.ShapeDtypeStruct((NUM_FACTORS, BATCH), jnp.float32),
            jax.ShapeDtypeStruct((NUM_FACTORS, BATCH), jnp.float32),
        ),
        scratch_types=[
            pltpu.VMEM((ROWBUF,), jnp.float32),
            pltpu.VMEM((CH,), jnp.int32),
            pltpu.VMEM((CH,), jnp.int32),
            pltpu.VMEM((CH,), jnp.float32),
            pltpu.VMEM((CH,), jnp.float32),
            pltpu.VMEM((NUM_FACTORS,), jnp.float32),
            pltpu.SemaphoreType.DMA,
            pltpu.SemaphoreType.DMA,
            pltpu.SemaphoreType.DMA,
            pltpu.SemaphoreType.DMA,
            pltpu.SemaphoreType.DMA,
            pltpu.SemaphoreType.DMA,
        ],
    )
    upart, ipart = p1(uft, ift, utail, itail, user_ids, item_ids, w_vec)

    p2 = pl.kernel(
        _pass2_kernel,
        mesh=mesh,
        compiler_params=_COMPILER_PARAMS,
        out_type=jax.ShapeDtypeStruct((BATCH,), jnp.float32),
        scratch_types=[
            pltpu.VMEM((NUM_FACTORS, B_PER_W), jnp.float32),
            pltpu.VMEM((NUM_FACTORS, B_PER_W), jnp.float32),
            pltpu.VMEM((L,), jnp.float32),
            pltpu.VMEM((B_PER_W,), jnp.float32),
            pltpu.SemaphoreType.DMA,
            pltpu.SemaphoreType.DMA,
            pltpu.SemaphoreType.DMA,
            pltpu.SemaphoreType.DMA,
        ],
    )
    return p2(upart, ipart, bias_splat)


def kernel(user_ids, item_ids, user_factors, item_factors, W, b):
    uid = user_ids.astype(jnp.int32)
    iid = item_ids.astype(jnp.int32)
    w_vec = W.reshape(NUM_FACTORS).astype(jnp.float32)
    bias_splat = jnp.broadcast_to(b.astype(jnp.float32), (L,))
    out = _run(uid, iid, user_factors, item_factors, w_vec, bias_splat)
    return out.reshape(BATCH, 1)


# speedup vs baseline: 5.0840x; 1.0428x over previous
import jax
import jax.numpy as jnp
from jax import lax
from jax.experimental import pallas as pl
from jax.experimental.pallas import tpu as pltpu
from jax.experimental.pallas import tpu_sc as plsc

NUM_FACTORS = 64
NUM_ROWS = 100000
BATCH = 16384
L = 16
NC = 2
NS = 16
NW = NC * NS
B_PER_W = BATCH // NW
MAIN = 99968
TAILPAD = 128
ROWBUF = MAIN + TAILPAD
CH = 4096
NCH = BATCH // CH

_COMPILER_PARAMS = pltpu.CompilerParams(
    needs_layout_passes=False, use_tc_tiling_on_sc=True)


def _pass1_kernel(uft_hbm, ift_hbm, utail_hbm, itail_hbm, uid_hbm, iid_hbm,
                  w_hbm, part_hbm,
                  rowbuf, prod_v, ids0_v, ids1_v, w_v,
                  sem_m, sem_t, sem_i0, sem_i1, sem_p):
    wid = lax.axis_index("s") * NC + lax.axis_index("c")
    pltpu.sync_copy(w_hbm, w_v)

    idbufs = [(ids0_v, sem_i0), (ids1_v, sem_i1)]

    def load_row(f, table, tail, ids_hbm):
        cm = pltpu.async_copy(table.at[f, pl.ds(0, MAIN)],
                              rowbuf.at[pl.ds(0, MAIN)], sem_m)
        ct = pltpu.async_copy(tail.at[f, :],
                              rowbuf.at[pl.ds(MAIN, TAILPAD)], sem_t)
        pltpu.async_copy(ids_hbm.at[pl.ds(0, CH)], idbufs[0][0],
                         idbufs[0][1])
        cm.wait()
        ct.wait()

    def chunks(ids_hbm, body):
        for h in range(NCH):
            ids_v, sem_i = idbufs[h % 2]
            if h + 1 < NCH:
                nxt_v, nxt_s = idbufs[(h + 1) % 2]
                pltpu.async_copy(ids_hbm.at[pl.ds((h + 1) * CH, CH)],
                                 nxt_v, nxt_s)
            pltpu.make_async_copy(ids_hbm.at[pl.ds(0, CH)], ids_v,
                                  sem_i).wait()
            body(h, ids_v)

    def factor(f, drain):
        load_row(f, uft_hbm, utail_hbm, uid_hbm)
        wspl = plsc.load_gather(w_v, [jnp.broadcast_to(f, (L,))])
        if drain:
            for _ in range(NCH):
                pltpu.make_async_copy(prod_v.at[pl.ds(0, CH)],
                                     part_hbm.at[0, pl.ds(0, CH)],
                                     sem_p).wait()

        def ubody(h, ids_v):
            @plsc.parallel_loop(0, CH // L, unroll=4)
            def _g(v):
                idxv = ids_v[pl.ds(v * L, L)]
                prod_v[pl.ds(h * CH + v * L, L)] = (
                    plsc.load_gather(rowbuf, [idxv]) * wspl)
        chunks(uid_hbm, ubody)

        load_row(f, ift_hbm, itail_hbm, iid_hbm)

        def ibody(h, ids_v):
            @plsc.parallel_loop(0, CH // L, unroll=4)
            def _g(v):
                idxv = ids_v[pl.ds(v * L, L)]
                o = h * CH + v * L
                prod_v[pl.ds(o, L)] = (prod_v[pl.ds(o, L)]
                                       * plsc.load_gather(rowbuf, [idxv]))
            pltpu.async_copy(prod_v.at[pl.ds(h * CH, CH)],
                             part_hbm.at[f, pl.ds(h * CH, CH)], sem_p)
        chunks(iid_hbm, ibody)

    factor(wid, False)
    factor(wid + 32, True)
    for _ in range(NCH):
        pltpu.make_async_copy(prod_v.at[pl.ds(0, CH)],
                              part_hbm.at[0, pl.ds(0, CH)], sem_p).wait()


def _pass2_kernel(part_hbm, bias_hbm, out_hbm,
                  ps_v, bias_v, out_v, sem_p0, sem_p1):
    wid = lax.axis_index("s") * NC + lax.axis_index("c")
    base = wid * B_PER_W
    hw = B_PER_W // 2
    c0 = pltpu.async_copy(part_hbm.at[:, pl.ds(base, hw)],
                          ps_v.at[:, pl.ds(0, hw)], sem_p0)
    c1 = pltpu.async_copy(part_hbm.at[:, pl.ds(base + hw, hw)],
                          ps_v.at[:, pl.ds(hw, hw)], sem_p1)
    pltpu.sync_copy(bias_hbm, bias_v)
    bias = bias_v[...]

    def slice_body(sl, carry):
        s0 = sl * L
        acc = bias
        for f in range(NUM_FACTORS):
            acc = acc + ps_v[f, pl.ds(s0, L)]
        out_v[pl.ds(s0, L)] = acc
        return carry

    c0.wait()
    lax.fori_loop(0, hw // L, slice_body, 0)
    c1.wait()
    lax.fori_loop(hw // L, B_PER_W // L, slice_body, 0)
    pltpu.sync_copy(out_v, out_hbm.at[pl.ds(base, B_PER_W)])


@jax.jit
def _run(user_ids, item_ids, user_factors, item_factors, w_vec, bias_splat):
    mesh = plsc.VectorSubcoreMesh(core_axis_name="c", subcore_axis_name="s")
    uft = user_factors.T
    ift = item_factors.T
    utail = jnp.pad(lax.slice(uft, (0, MAIN), (NUM_FACTORS, NUM_ROWS)),
                    ((0, 0), (0, TAILPAD - (NUM_ROWS - MAIN))))
    itail = jnp.pad(lax.slice(ift, (0, MAIN), (NUM_FACTORS, NUM_ROWS)),
                    ((0, 0), (0, TAILPAD - (NUM_ROWS - MAIN))))

    p1 = pl.kernel(
        _pass1_kernel,
        mesh=mesh,
        compiler_params=_COMPILER_PARAMS,
        out_type=jax.ShapeDtypeStruct((NUM_FACTORS, BATCH), jnp.float32),
        scratch_types=[
            pltpu.VMEM((ROWBUF,), jnp.float32),
            pltpu.VMEM((BATCH,), jnp.float32),
            pltpu.VMEM((CH,), jnp.int32),
            pltpu.VMEM((CH,), jnp.int32),
            pltpu.VMEM((NUM_FACTORS,), jnp.float32),
            pltpu.SemaphoreType.DMA,
            pltpu.SemaphoreType.DMA,
            pltpu.SemaphoreType.DMA,
            pltpu.SemaphoreType.DMA,
            pltpu.SemaphoreType.DMA,
        ],
    )
    part = p1(uft, ift, utail, itail, user_ids, item_ids, w_vec)

    p2 = pl.kernel(
        _pass2_kernel,
        mesh=mesh,
        compiler_params=_COMPILER_PARAMS,
        out_type=jax.ShapeDtypeStruct((BATCH,), jnp.float32),
        scratch_types=[
            pltpu.VMEM((NUM_FACTORS, B_PER_W), jnp.float32),
            pltpu.VMEM((L,), jnp.float32),
            pltpu.VMEM((B_PER_W,), jnp.float32),
            pltpu.SemaphoreType.DMA,
            pltpu.SemaphoreType.DMA,
        ],
    )
    return p2(part, bias_splat)


def kernel(user_ids, item_ids, user_factors, item_factors, W, b):
    uid = user_ids.astype(jnp.int32)
    iid = item_ids.astype(jnp.int32)
    w_vec = W.reshape(NUM_FACTORS).astype(jnp.float32)
    bias_splat = jnp.broadcast_to(b.astype(jnp.float32), (L,))
    out = _run(uid, iid, user_factors, item_factors, w_vec, bias_splat)
    return out.reshape(BATCH, 1)


# speedup vs baseline: 5.8770x; 1.1560x over previous
import jax
import jax.numpy as jnp
from jax import lax
from jax.experimental import pallas as pl
from jax.experimental.pallas import tpu as pltpu
from jax.experimental.pallas import tpu_sc as plsc

NUM_FACTORS = 64
NUM_ROWS = 100000
BATCH = 16384
L = 16
NC = 2
NS = 16
NW = NC * NS
MAIN = 99968
TAILPAD = 128
ROWBUF = MAIN + TAILPAD
CH = 4096
NCH = BATCH // CH
PR = 128
CROWS = CH // 128

_COMPILER_PARAMS = pltpu.CompilerParams(
    needs_layout_passes=False, use_tc_tiling_on_sc=True)


def _pass1_kernel(uft_hbm, ift_hbm, utail_hbm, itail_hbm, uid_hbm, iid_hbm,
                  w_hbm, accs_hbm,
                  rowbuf, prod_v, ids0_v, ids1_v, w_v, zbuf_v, idxrows_v,
                  acc_sh, sem_m, sem_t, sem_i0, sem_i1):
    sid = lax.axis_index("s")
    cid = lax.axis_index("c")
    wid = sid * NC + cid
    pltpu.sync_copy(w_hbm, w_v)

    zero = jnp.zeros((L,), jnp.float32)
    iota = lax.iota(jnp.int32, L)
    for r in range(8):
        for c in range(8):
            zbuf_v[r, pl.ds(c * L, L)] = zero
    for h in range(NCH):
        for j in range(2):
            idxrows_v[h, pl.ds(j * L, L)] = iota + (h * CROWS + j * L)
    pltpu.sync_copy(zbuf_v, acc_sh.at[pl.ds(sid * 8, 8), :])
    plsc.subcore_barrier()

    idbufs = [(ids0_v, sem_i0), (ids1_v, sem_i1)]

    def load_row(f, table, tail, ids_hbm):
        cm = pltpu.async_copy(table.at[f, pl.ds(0, MAIN)],
                              rowbuf.at[pl.ds(0, MAIN)], sem_m)
        ct = pltpu.async_copy(tail.at[f, :],
                              rowbuf.at[pl.ds(MAIN, TAILPAD)], sem_t)
        pltpu.async_copy(ids_hbm.at[pl.ds(0, CH)], idbufs[0][0],
                         idbufs[0][1])
        cm.wait()
        ct.wait()

    def chunks(ids_hbm, body):
        for h in range(NCH):
            ids_v, sem_i = idbufs[h % 2]
            if h + 1 < NCH:
                nxt_v, nxt_s = idbufs[(h + 1) % 2]
                pltpu.async_copy(ids_hbm.at[pl.ds((h + 1) * CH, CH)],
                                 nxt_v, nxt_s)
            pltpu.make_async_copy(ids_hbm.at[pl.ds(0, CH)], ids_v,
                                  sem_i).wait()
            body(h, ids_v)

    def factor(f):
        load_row(f, uft_hbm, utail_hbm, uid_hbm)
        wspl = plsc.load_gather(w_v, [jnp.broadcast_to(f, (L,))])

        def ubody(h, ids_v):
            @plsc.parallel_loop(0, CH // L, unroll=4)
            def _g(v):
                idxv = ids_v[pl.ds(v * L, L)]
                r = h * CROWS + lax.shift_right_logical(v, 3)
                c = (v & 7) * L
                prod_v[r, pl.ds(c, L)] = (
                    plsc.load_gather(rowbuf, [idxv]) * wspl)
        chunks(uid_hbm, ubody)

        load_row(f, ift_hbm, itail_hbm, iid_hbm)

        def ibody(h, ids_v):
            @plsc.parallel_loop(0, CH // L, unroll=4)
            def _g(v):
                idxv = ids_v[pl.ds(v * L, L)]
                r = h * CROWS + lax.shift_right_logical(v, 3)
                c = (v & 7) * L
                prod_v[r, pl.ds(c, L)] = (
                    prod_v[r, pl.ds(c, L)]
                    * plsc.load_gather(rowbuf, [idxv]))
            pltpu.sync_copy(prod_v.at[pl.ds(h * CROWS, CROWS), :],
                            acc_sh.at[idxrows_v.at[h]], add=True)
        chunks(iid_hbm, ibody)

    factor(wid)
    factor(wid + 32)

    plsc.subcore_barrier()

    @pl.when(sid == 0)
    def _writeout():
        pltpu.sync_copy(acc_sh, accs_hbm.at[cid])


@jax.jit
def _run(user_ids, item_ids, user_factors, item_factors, w_vec, b):
    mesh = plsc.VectorSubcoreMesh(core_axis_name="c", subcore_axis_name="s")
    uft = user_factors.T
    ift = item_factors.T
    utail = jnp.pad(lax.slice(uft, (0, MAIN), (NUM_FACTORS, NUM_ROWS)),
                    ((0, 0), (0, TAILPAD - (NUM_ROWS - MAIN))))
    itail = jnp.pad(lax.slice(ift, (0, MAIN), (NUM_FACTORS, NUM_ROWS)),
                    ((0, 0), (0, TAILPAD - (NUM_ROWS - MAIN))))

    p1 = pl.kernel(
        _pass1_kernel,
        mesh=mesh,
        compiler_params=_COMPILER_PARAMS,
        out_type=jax.ShapeDtypeStruct((NC, PR, 128), jnp.float32),
        scratch_types=[
            pltpu.VMEM((ROWBUF,), jnp.float32),
            pltpu.VMEM((PR, 128), jnp.float32),
            pltpu.VMEM((CH,), jnp.int32),
            pltpu.VMEM((CH,), jnp.int32),
            pltpu.VMEM((NUM_FACTORS,), jnp.float32),
            pltpu.VMEM((8, 128), jnp.float32),
            pltpu.VMEM((NCH, CROWS), jnp.int32),
            pltpu.VMEM_SHARED((PR, 128), jnp.float32),
            pltpu.SemaphoreType.DMA,
            pltpu.SemaphoreType.DMA,
            pltpu.SemaphoreType.DMA,
            pltpu.SemaphoreType.DMA,
        ],
    )
    accs = p1(uft, ift, utail, itail, user_ids, item_ids, w_vec)
    return (accs[0] + accs[1]).reshape(BATCH) + b[0]


def kernel(user_ids, item_ids, user_factors, item_factors, W, b):
    uid = user_ids.astype(jnp.int32)
    iid = item_ids.astype(jnp.int32)
    w_vec = W.reshape(NUM_FACTORS).astype(jnp.float32)
    out = _run(uid, iid, user_factors, item_factors, w_vec,
               b.astype(jnp.float32))
    return out.reshape(BATCH, 1)
